# Initial kernel scaffold; baseline (speedup 1.0000x reference)
import jax
import jax.numpy as jnp
from jax.experimental import pallas as pl


def kernel(continuous_idx, W):
    raise NotImplementedError("write your pallas kernel here")


# speedup vs baseline: 2.2615x; 2.2615x over previous
import functools

import jax
import jax.numpy as jnp
from jax import lax
from jax.experimental import pallas as pl
from jax.experimental.pallas import tpu as pltpu
from jax.experimental.pallas import tpu_sc as plsc

L = 16
NC, NS = 2, 16
NW = NC * NS
CHUNK = 1024
IDXROW = 128
KSTREAM = CHUNK // IDXROW


@functools.lru_cache(maxsize=None)
def _build(n_total, vocab, dim):
    per_w = n_total // NW
    n_chunks = per_w // CHUNK
    max_idx = vocab - 1

    mesh = plsc.VectorSubcoreMesh(core_axis_name="c", subcore_axis_name="s")

    @functools.partial(
        pl.kernel,
        out_type=jax.ShapeDtypeStruct((n_total, dim), jnp.float32),
        mesh=mesh,
        compiler_params=pltpu.CompilerParams(use_tc_tiling_on_sc=False),
        scratch_types=[
            pltpu.VMEM((CHUNK,), jnp.float32),
            pltpu.VMEM((KSTREAM, IDXROW), jnp.int32),
            pltpu.VMEM((KSTREAM, IDXROW), jnp.int32),
            pltpu.VMEM((CHUNK, dim), jnp.float32),
            pltpu.VMEM((CHUNK, dim), jnp.float32),
            pltpu.SemaphoreType.DMA,
        ],
    )
    def impl(cont_hbm, w_hbm, out_hbm, cont_v, idx_lo_v, idx_hi_v, lo_v, hi_v, sem):
        wid = lax.axis_index("s") * NC + lax.axis_index("c")

        def chunk_body(g, _):
            base = wid * per_w + g * CHUNK
            pltpu.sync_copy(cont_hbm.at[pl.ds(base, CHUNK)], cont_v)

            def idx_body(t, _):
                x = cont_v[pl.ds(t * L, L)]
                il = x.astype(jnp.int32)
                w = x - il.astype(jnp.float32)
                ih = jnp.minimum(il + 1, max_idx)
                r = t // (IDXROW // L)
                c = (t % (IDXROW // L)) * L
                idx_lo_v[r, pl.ds(c, L)] = il
                idx_hi_v[r, pl.ds(c, L)] = ih
                cont_v[pl.ds(t * L, L)] = w
                return 0

            lax.fori_loop(0, CHUNK // L, idx_body, 0)

            copies = []
            for j in range(KSTREAM):
                copies.append(pltpu.async_copy(
                    w_hbm.at[idx_lo_v.at[j]], lo_v.at[pl.ds(j * IDXROW, IDXROW)], sem))
                copies.append(pltpu.async_copy(
                    w_hbm.at[idx_hi_v.at[j]], hi_v.at[pl.ds(j * IDXROW, IDXROW)], sem))
            for cp in copies:
                cp.wait()

            def blend_body(t, _):
                w16 = cont_v[pl.ds(t * L, L)]
                for k in range(L):
                    i = t * L + k
                    wv = lax.gather(
                        w16, jnp.full((L, 1), k, jnp.int32),
                        lax.GatherDimensionNumbers(
                            offset_dims=(), collapsed_slice_dims=(0,),
                            start_index_map=(0,)),
                        slice_sizes=(1,),
                        mode=lax.GatherScatterMode.PROMISE_IN_BOUNDS)
                    for d in range(dim // L):
                        lo = lo_v[i, pl.ds(d * L, L)]
                        hi = hi_v[i, pl.ds(d * L, L)]
                        lo_v[i, pl.ds(d * L, L)] = lo + wv * (hi - lo)
                return 0

            lax.fori_loop(0, CHUNK // L, blend_body, 0)

            pltpu.sync_copy(lo_v, out_hbm.at[pl.ds(base, CHUNK)])
            return 0

        lax.fori_loop(0, n_chunks, chunk_body, 0)

    return impl


def kernel(continuous_idx, W):
    batch, fields = continuous_idx.shape
    vocab, dim = W.shape
    n_total = batch * fields
    impl = _build(n_total, vocab, dim)
    out = impl(continuous_idx.reshape(n_total), W)
    return out.reshape(batch, fields, dim)
